# Initial kernel scaffold; baseline (speedup 1.0000x reference)
import jax
import jax.numpy as jnp
from jax.experimental import pallas as pl


def kernel(x, edge_index, w_mul, W1, a1_src, a1_dst, W2, a2_src, a2_dst):
    raise NotImplementedError("write your pallas kernel here")


# speedup vs baseline: 29.8416x; 29.8416x over previous
import functools

import jax
import jax.numpy as jnp
from jax import lax
from jax.experimental import pallas as pl
from jax.experimental.pallas import tpu as pltpu
from jax.experimental.pallas import tpu_sc as plsc

N = 10000
NP = 10240
D = 128
H1 = 8
O1 = 8
F1 = H1 * O1
C = 16
E = 320000
NC = 2
NS = 16
ROW = 16
BN = 512

CH1 = 4
EPT1 = E // CH1
B1 = 64
NB1 = EPT1 // B1

CH2 = 16
EPT2 = E // CH2
B2 = 80
NB2 = EPT2 // B2


def _k1_body(x_ref, w1_ref, a1s_ref, a1d_ref, h_ref, ssrc_ref, sdst_ref):
    xb = x_ref[...]
    for h in range(H1):
        hh = lax.dot_general(xb, w1_ref[:, h * O1:(h + 1) * O1],
                             (((1,), (0,)), ((), ())),
                             preferred_element_type=jnp.float32)
        h_ref[h] = hh
        ssrc_ref[h] = jnp.sum(hh * a1s_ref[h][None, :], axis=1)
        sdst_ref[h] = jnp.sum(hh * a1d_ref[h][None, :], axis=1)


_k1 = pl.pallas_call(
    _k1_body,
    grid=(NP // BN,),
    in_specs=[
        pl.BlockSpec((BN, D), lambda i: (i, 0)),
        pl.BlockSpec((D, F1), lambda i: (0, 0)),
        pl.BlockSpec((H1, O1), lambda i: (0, 0)),
        pl.BlockSpec((H1, O1), lambda i: (0, 0)),
    ],
    out_specs=[
        pl.BlockSpec((H1, BN, O1), lambda i: (0, i, 0)),
        pl.BlockSpec((H1, BN), lambda i: (0, i)),
        pl.BlockSpec((H1, BN), lambda i: (0, i)),
    ],
    out_shape=[
        jax.ShapeDtypeStruct((H1, NP, O1), jnp.float32),
        jax.ShapeDtypeStruct((H1, NP), jnp.float32),
        jax.ShapeDtypeStruct((H1, NP), jnp.float32),
    ],
)


def _edge_pass(ei, wm, htab, stab_src, stab_dst, out,
               ssrc_tab, sdst_tab, srcb, dstb, wb, gidxb, rowsb,
               rowb, idxb, zb, semi0, semi1, semg0, semg1, acc,
               *, blk_e, n_blk, n_heads, n_chunk_per_core, row_w,
               add_head_off, store_q, acc_rows):
    c = lax.axis_index("c")
    s = lax.axis_index("s")
    head = s % n_heads
    chunk = c * n_chunk_per_core + s // n_heads
    semi = (semi0, semi1)
    semg = (semg0, semg1)

    pltpu.sync_copy(stab_src.at[head], ssrc_tab)
    pltpu.sync_copy(stab_dst.at[head], sdst_tab)

    for i in range(128):
        zb[i, :] = jnp.zeros((ROW,), jnp.float32)
    rows_per_tile = acc_rows // NS
    row0 = s * rows_per_tile

    def _zero(j, carry):
        pltpu.sync_copy(zb, acc.at[pl.ds(row0 + j * 128, 128)])
        return carry

    lax.fori_loop(0, rows_per_tile // 128, _zero, 0)
    for i in range(blk_e):
        rowb[i, :] = jnp.zeros((ROW,), jnp.float32)
    plsc.subcore_barrier()

    ebase = chunk * (blk_e * n_blk)
    lanes = lax.iota(jnp.int32, 16)
    n_groups = blk_e // 16

    def _in_dma(blk, buf):
        off = pl.multiple_of(ebase + blk * blk_e, 8)
        pltpu.async_copy(ei.at[0, pl.ds(off, blk_e)], srcb.at[buf], semi[buf])
        pltpu.async_copy(ei.at[1, pl.ds(off, blk_e)], dstb.at[buf], semi[buf])
        pltpu.async_copy(wm.at[pl.ds(off, blk_e)], wb.at[buf], semi[buf])

    def _wait_in(buf):
        pltpu.make_async_copy(ei.at[0, pl.ds(0, blk_e)], srcb.at[buf],
                              semi[buf]).wait()
        pltpu.make_async_copy(ei.at[1, pl.ds(0, blk_e)], dstb.at[buf],
                              semi[buf]).wait()
        pltpu.make_async_copy(wm.at[pl.ds(0, blk_e)], wb.at[buf],
                              semi[buf]).wait()

    def _gather(buf):
        if add_head_off:
            for g in range(n_groups):
                sl = pl.ds(g * 16, 16)
                gidxb[buf, sl] = srcb[buf, sl] + head * NP
            idx_ref = gidxb.at[buf]
        else:
            idx_ref = srcb.at[buf]
        pltpu.async_copy(htab.at[idx_ref], rowsb.at[buf], semg[buf])

    def _wait_gather(buf):
        if add_head_off:
            idx_ref = gidxb.at[buf]
        else:
            idx_ref = srcb.at[buf]
        pltpu.make_async_copy(htab.at[idx_ref], rowsb.at[buf],
                              semg[buf]).wait()

    def _compute(buf):
        col0 = head * O1 if not add_head_off else 0
        for g in range(n_groups):
            sl = pl.ds(g * 16, 16)
            srcv = srcb[buf, sl]
            dstv = dstb[buf, sl]
            wv = wb[buf, sl]
            a = plsc.load_gather(ssrc_tab, [srcv])
            bb = plsc.load_gather(sdst_tab, [dstv])
            e = a + bb
            e = jnp.maximum(e, 0.2 * e)
            p = jnp.exp(e)
            q = p * wv
            rid = g * 16 + lanes
            for o in range(O1):
                hv = plsc.load_gather(
                    rowsb.at[buf], [rid, jnp.full((16,), o, jnp.int32) + col0])
                plsc.store_scatter(rowb, [rid, jnp.full((16,), o, jnp.int32)],
                                   hv * q)
            if store_q:
                plsc.store_scatter(rowb, [rid, jnp.full((16,), 8, jnp.int32)],
                                   q)
                plsc.store_scatter(rowb, [rid, jnp.full((16,), 9, jnp.int32)],
                                   p)
            else:
                plsc.store_scatter(rowb, [rid, jnp.full((16,), 8, jnp.int32)],
                                   p)
            idxb[sl] = dstv + head * NP

    _in_dma(0, 0)
    _wait_in(0)
    _gather(0)
    _in_dma(1, 1)

    def _pair(i, carry):
        for b in (0, 1):
            blk = 2 * i + b
            _wait_gather(b)

            @pl.when(blk + 1 < n_blk)
            def _():
                _wait_in(1 - b)
                _gather(1 - b)

            _compute(b)
            pltpu.sync_copy(rowb, acc.at[idxb], add=True)

            @pl.when(blk + 2 < n_blk)
            def _():
                _in_dma(blk + 2, b)
        return carry

    lax.fori_loop(0, n_blk // 2, _pair, 0)

    plsc.subcore_barrier()
    pltpu.sync_copy(acc.at[pl.ds(row0, rows_per_tile)],
                    out.at[c, pl.ds(row0, rows_per_tile)])


def _make_edge_kernel(n_heads, n_chunk_per_core, blk_e, n_blk, row_w,
                      add_head_off, store_q, acc_rows):
    body = functools.partial(
        _edge_pass,
        blk_e=blk_e, n_blk=n_blk, n_heads=n_heads,
        n_chunk_per_core=n_chunk_per_core, row_w=row_w,
        add_head_off=add_head_off, store_q=store_q, acc_rows=acc_rows,
    )

    return pl.kernel(
        body,
        out_type=jax.ShapeDtypeStruct((NC, acc_rows, ROW), jnp.float32),
        mesh=plsc.VectorSubcoreMesh(core_axis_name="c", subcore_axis_name="s",
                                    num_cores=NC, num_subcores=NS),
        compiler_params=pltpu.CompilerParams(needs_layout_passes=False,
                                             use_tc_tiling_on_sc=False),
        scratch_types=[
            pltpu.VMEM((NP,), jnp.float32),
            pltpu.VMEM((NP,), jnp.float32),
            pltpu.VMEM((2, blk_e), jnp.int32),
            pltpu.VMEM((2, blk_e), jnp.int32),
            pltpu.VMEM((2, blk_e), jnp.float32),
            pltpu.VMEM((2, blk_e), jnp.int32),
            pltpu.VMEM((2, blk_e, row_w), jnp.float32),
            pltpu.VMEM((blk_e, ROW), jnp.float32),
            pltpu.VMEM((blk_e,), jnp.int32),
            pltpu.VMEM((128, ROW), jnp.float32),
            pltpu.SemaphoreType.DMA,
            pltpu.SemaphoreType.DMA,
            pltpu.SemaphoreType.DMA,
            pltpu.SemaphoreType.DMA,
            pltpu.VMEM_SHARED((acc_rows, ROW), jnp.float32),
        ],
    )


_edge_kernel_cache = {}


def _get_edge_kernel(which):
    if which not in _edge_kernel_cache:
        if which == "l1":
            _edge_kernel_cache[which] = _make_edge_kernel(
                H1, CH1 // NC, B1, NB1, O1, True, True, H1 * NP)
        else:
            _edge_kernel_cache[which] = _make_edge_kernel(
                2, CH2 // NC, B2, NB2, C, False, False, 2 * NP)
    return _edge_kernel_cache[which]


def _k3_body(acc_ref, w2r_ref, a2_ref, h2_ref, s2_ref):
    h2 = jnp.zeros((BN, C), jnp.float32)
    for h in range(H1):
        y = acc_ref[0, h] + acc_ref[1, h]
        inv = 1.0 / (y[:, 9] + 1e-16)
        aggh = y[:, 0:O1] * inv[:, None]
        curvh = y[:, 8] * inv
        aggh = jnp.where(aggh > 0, aggh, jnp.exp(aggh) - 1.0)
        curvh = jnp.where(curvh > 0, curvh, jnp.exp(curvh) - 1.0)
        h2 = h2 + lax.dot_general(aggh, w2r_ref[h, 0:O1, :],
                                  (((1,), (0,)), ((), ())),
                                  preferred_element_type=jnp.float32)
        h2 = h2 + curvh[:, None] * w2r_ref[h, O1, :][None, :]
    h2_ref[...] = h2
    s2_ref[...] = lax.dot_general(a2_ref[...], h2, (((1,), (1,)), ((), ())),
                                  preferred_element_type=jnp.float32)


_k3 = pl.pallas_call(
    _k3_body,
    grid=(NP // BN,),
    in_specs=[
        pl.BlockSpec((NC, H1, BN, ROW), lambda i: (0, 0, i, 0)),
        pl.BlockSpec((H1, O1 + 1, C), lambda i: (0, 0, 0)),
        pl.BlockSpec((H1, C), lambda i: (0, 0)),
    ],
    out_specs=[
        pl.BlockSpec((BN, C), lambda i: (i, 0)),
        pl.BlockSpec((H1, BN), lambda i: (0, i)),
    ],
    out_shape=[
        jax.ShapeDtypeStruct((NP, C), jnp.float32),
        jax.ShapeDtypeStruct((H1, NP), jnp.float32),
    ],
)


def _k5_body(acc_ref, out_ref):
    y = acc_ref[0] + acc_ref[1]
    inv = 1.0 / (y[0, :, 8] + 1e-16)
    x2 = jnp.concatenate([y[0, :, 0:O1], y[1, :, 0:O1]], axis=1)
    x2 = x2 * inv[:, None]
    m = jnp.max(x2, axis=1, keepdims=True)
    z = x2 - m
    out_ref[...] = z - jnp.log(jnp.sum(jnp.exp(z), axis=1, keepdims=True))


_k5 = pl.pallas_call(
    _k5_body,
    grid=(NP // BN,),
    in_specs=[pl.BlockSpec((NC, 2, BN, ROW), lambda i: (0, 0, i, 0))],
    out_specs=pl.BlockSpec((BN, C), lambda i: (i, 0)),
    out_shape=jax.ShapeDtypeStruct((NP, C), jnp.float32),
)


def kernel(x, edge_index, w_mul, W1, a1_src, a1_dst, W2, a2_src, a2_dst):
    xp = jnp.pad(x, ((0, NP - N), (0, 0)))

    h1hm, ssrct, sdstt = _k1(xp, W1, a1_src, a1_dst)
    acc1 = _get_edge_kernel("l1")(edge_index, w_mul,
                                  h1hm.reshape(H1 * NP, O1), ssrct, sdstt)

    w2r = W2.reshape(H1, O1 + 1, C)
    a2 = jnp.concatenate(
        [a2_src, a2_dst, jnp.zeros((H1 - 2, C), jnp.float32)], axis=0)
    h2, s2 = _k3(acc1.reshape(NC, H1, NP, ROW), w2r, a2)

    s2src = jnp.broadcast_to(s2[0:1], (2, NP))
    s2dst = jnp.broadcast_to(s2[1:2], (2, NP))
    acc2 = _get_edge_kernel("l2")(edge_index, w_mul, h2, s2src, s2dst)

    out = _k5(acc2.reshape(NC, 2, NP, ROW))
    return out[:N]


# speedup vs baseline: 32.2387x; 1.0803x over previous
import functools

import jax
import jax.numpy as jnp
from jax import lax
from jax.experimental import pallas as pl
from jax.experimental.pallas import tpu as pltpu
from jax.experimental.pallas import tpu_sc as plsc

N = 10000
NP = 10240
D = 128
H1 = 8
O1 = 8
F1 = H1 * O1
C = 16
E = 320000
NC = 2
NS = 16
RW1 = 11
RW2 = 9
BN = 512

CH1 = 4
EPT1 = E // CH1
B1 = 64
NB1 = EPT1 // B1

CH2 = 16
EPT2 = E // CH2
B2 = 80
NB2 = EPT2 // B2


def _k1_body(x_ref, w1_ref, a1s_ref, a1d_ref, h_ref, ssrc_ref, sdst_ref):
    xb = x_ref[...]
    for h in range(H1):
        hh = lax.dot_general(xb, w1_ref[:, h * O1:(h + 1) * O1],
                             (((1,), (0,)), ((), ())),
                             preferred_element_type=jnp.float32)
        h_ref[h] = hh
        ssrc_ref[h] = jnp.sum(hh * a1s_ref[h][None, :], axis=1)
        sdst_ref[h] = jnp.sum(hh * a1d_ref[h][None, :], axis=1)


_k1 = pl.pallas_call(
    _k1_body,
    grid=(NP // BN,),
    in_specs=[
        pl.BlockSpec((BN, D), lambda i: (i, 0)),
        pl.BlockSpec((D, F1), lambda i: (0, 0)),
        pl.BlockSpec((H1, O1), lambda i: (0, 0)),
        pl.BlockSpec((H1, O1), lambda i: (0, 0)),
    ],
    out_specs=[
        pl.BlockSpec((H1, BN, O1), lambda i: (0, i, 0)),
        pl.BlockSpec((H1, BN), lambda i: (0, i)),
        pl.BlockSpec((H1, BN), lambda i: (0, i)),
    ],
    out_shape=[
        jax.ShapeDtypeStruct((H1, NP, O1), jnp.float32),
        jax.ShapeDtypeStruct((H1, NP), jnp.float32),
        jax.ShapeDtypeStruct((H1, NP), jnp.float32),
    ],
)


def _edge_pass(ei, wm, htab, stab_src, stab_dst, zhbm, out,
               ssrc_tab, sdst_tab, srcb, dstb, wb, gidxb, rowsb,
               rowb, idxb0, idxb1,
               semi0, semi1, semg0, semg1, sems0, sems1, acc,
               *, blk_e, n_blk, n_heads, n_chunk_per_core, gat_w, rw,
               add_head_off, store_q, acc_rows):
    c = lax.axis_index("c")
    s = lax.axis_index("s")
    head = s % n_heads
    chunk = c * n_chunk_per_core + s // n_heads
    semi = (semi0, semi1)
    semg = (semg0, semg1)
    sems = (sems0, sems1)
    idxb = (idxb0, idxb1)

    pltpu.sync_copy(stab_src.at[head], ssrc_tab)
    pltpu.sync_copy(stab_dst.at[head], sdst_tab)

    rows_per_tile = acc_rows // NS
    row0 = s * rows_per_tile
    pltpu.sync_copy(zhbm, acc.at[pl.ds(row0, rows_per_tile)])

    ebase = chunk * (blk_e * n_blk)
    lanes = lax.iota(jnp.int32, 16)
    n_groups = blk_e // 16

    for buf in range(2):
        for col in range(10 if store_q else 9, rw):
            for g in range(n_groups):
                plsc.store_scatter(rowb.at[buf],
                                   [g * 16 + lanes,
                                    jnp.full((16,), col, jnp.int32)],
                                   jnp.zeros((16,), jnp.float32))
    plsc.subcore_barrier()

    def _in_dma(blk, buf):
        off = pl.multiple_of(ebase + blk * blk_e, 8)
        pltpu.async_copy(ei.at[0, pl.ds(off, blk_e)], srcb.at[buf], semi[buf])
        pltpu.async_copy(ei.at[1, pl.ds(off, blk_e)], dstb.at[buf], semi[buf])
        pltpu.async_copy(wm.at[pl.ds(off, blk_e)], wb.at[buf], semi[buf])

    def _wait_in(buf):
        pltpu.make_async_copy(ei.at[0, pl.ds(0, blk_e)], srcb.at[buf],
                              semi[buf]).wait()
        pltpu.make_async_copy(ei.at[1, pl.ds(0, blk_e)], dstb.at[buf],
                              semi[buf]).wait()
        pltpu.make_async_copy(wm.at[pl.ds(0, blk_e)], wb.at[buf],
                              semi[buf]).wait()

    def _gather(buf):
        if add_head_off:
            for g in range(n_groups):
                sl = pl.ds(g * 16, 16)
                gidxb[buf, sl] = srcb[buf, sl] + head * NP
            idx_ref = gidxb.at[buf]
        else:
            idx_ref = srcb.at[buf]
        pltpu.async_copy(htab.at[idx_ref], rowsb.at[buf], semg[buf])

    def _wait_gather(buf):
        if add_head_off:
            idx_ref = gidxb.at[buf]
        else:
            idx_ref = srcb.at[buf]
        pltpu.make_async_copy(htab.at[idx_ref], rowsb.at[buf],
                              semg[buf]).wait()

    def _scat(buf):
        pltpu.async_copy(rowb.at[buf], acc.at[idxb[buf]], sems[buf],
                         add=True)

    def _wait_scat(buf):
        pltpu.make_async_copy(rowb.at[buf], acc.at[idxb[buf]],
                              sems[buf]).wait()

    def _compute(buf):
        col0 = head * O1 if not add_head_off else 0
        for g in range(n_groups):
            sl = pl.ds(g * 16, 16)
            srcv = srcb[buf, sl]
            dstv = dstb[buf, sl]
            wv = wb[buf, sl]
            a = plsc.load_gather(ssrc_tab, [srcv])
            bb = plsc.load_gather(sdst_tab, [dstv])
            e = a + bb
            e = jnp.maximum(e, 0.2 * e)
            p = jnp.exp(e)
            q = p * wv
            rid = g * 16 + lanes
            for o in range(O1):
                hv = plsc.load_gather(
                    rowsb.at[buf], [rid, jnp.full((16,), o, jnp.int32) + col0])
                plsc.store_scatter(rowb.at[buf],
                                   [rid, jnp.full((16,), o, jnp.int32)],
                                   hv * q)
            if store_q:
                plsc.store_scatter(rowb.at[buf],
                                   [rid, jnp.full((16,), 8, jnp.int32)], q)
                plsc.store_scatter(rowb.at[buf],
                                   [rid, jnp.full((16,), 9, jnp.int32)], p)
            else:
                plsc.store_scatter(rowb.at[buf],
                                   [rid, jnp.full((16,), 8, jnp.int32)], p)
            idxb[buf][sl] = dstv + head * NP

    _in_dma(0, 0)
    _wait_in(0)
    _gather(0)
    _in_dma(1, 1)

    def _pair(i, carry):
        for b in (0, 1):
            blk = 2 * i + b
            _wait_gather(b)

            @pl.when(blk + 1 < n_blk)
            def _():
                _wait_in(1 - b)
                _gather(1 - b)

            @pl.when(blk >= 2)
            def _():
                _wait_scat(b)

            _compute(b)
            _scat(b)

            @pl.when(blk + 2 < n_blk)
            def _():
                _in_dma(blk + 2, b)
        return carry

    lax.fori_loop(0, n_blk // 2, _pair, 0)

    _wait_scat(0)
    _wait_scat(1)
    plsc.subcore_barrier()
    pltpu.sync_copy(acc.at[pl.ds(row0, rows_per_tile)],
                    out.at[c, pl.ds(row0, rows_per_tile)])


def _make_edge_kernel(n_heads, n_chunk_per_core, blk_e, n_blk, gat_w, rw,
                      add_head_off, store_q, acc_rows):
    body = functools.partial(
        _edge_pass,
        blk_e=blk_e, n_blk=n_blk, n_heads=n_heads,
        n_chunk_per_core=n_chunk_per_core, gat_w=gat_w, rw=rw,
        add_head_off=add_head_off, store_q=store_q, acc_rows=acc_rows,
    )

    return pl.kernel(
        body,
        out_type=jax.ShapeDtypeStruct((NC, acc_rows, rw), jnp.float32),
        mesh=plsc.VectorSubcoreMesh(core_axis_name="c", subcore_axis_name="s",
                                    num_cores=NC, num_subcores=NS),
        compiler_params=pltpu.CompilerParams(needs_layout_passes=False,
                                             use_tc_tiling_on_sc=False),
        scratch_types=[
            pltpu.VMEM((NP,), jnp.float32),
            pltpu.VMEM((NP,), jnp.float32),
            pltpu.VMEM((2, blk_e), jnp.int32),
            pltpu.VMEM((2, blk_e), jnp.int32),
            pltpu.VMEM((2, blk_e), jnp.float32),
            pltpu.VMEM((2, blk_e), jnp.int32),
            pltpu.VMEM((2, blk_e, gat_w), jnp.float32),
            pltpu.VMEM((2, blk_e, rw), jnp.float32),
            pltpu.VMEM((blk_e,), jnp.int32),
            pltpu.VMEM((blk_e,), jnp.int32),
            pltpu.SemaphoreType.DMA,
            pltpu.SemaphoreType.DMA,
            pltpu.SemaphoreType.DMA,
            pltpu.SemaphoreType.DMA,
            pltpu.SemaphoreType.DMA,
            pltpu.SemaphoreType.DMA,
            pltpu.VMEM_SHARED((acc_rows, rw), jnp.float32),
        ],
    )


_edge_kernel_cache = {}


def _get_edge_kernel(which):
    if which not in _edge_kernel_cache:
        if which == "l1":
            _edge_kernel_cache[which] = _make_edge_kernel(
                H1, CH1 // NC, B1, NB1, O1, RW1, True, True, H1 * NP)
        else:
            _edge_kernel_cache[which] = _make_edge_kernel(
                2, CH2 // NC, B2, NB2, C, RW2, False, False, 2 * NP)
    return _edge_kernel_cache[which]


def _k3_body(acc_ref, w2r_ref, a2_ref, h2_ref, s2_ref):
    h2 = jnp.zeros((BN, C), jnp.float32)
    for h in range(H1):
        y = acc_ref[0, h] + acc_ref[1, h]
        inv = 1.0 / (y[:, 9] + 1e-16)
        aggh = y[:, 0:O1] * inv[:, None]
        curvh = y[:, 8] * inv
        aggh = jnp.where(aggh > 0, aggh, jnp.exp(aggh) - 1.0)
        curvh = jnp.where(curvh > 0, curvh, jnp.exp(curvh) - 1.0)
        h2 = h2 + lax.dot_general(aggh, w2r_ref[h, 0:O1, :],
                                  (((1,), (0,)), ((), ())),
                                  preferred_element_type=jnp.float32)
        h2 = h2 + curvh[:, None] * w2r_ref[h, O1, :][None, :]
    h2_ref[...] = h2
    s2_ref[...] = lax.dot_general(a2_ref[...], h2, (((1,), (1,)), ((), ())),
                                  preferred_element_type=jnp.float32)


_k3 = pl.pallas_call(
    _k3_body,
    grid=(NP // BN,),
    in_specs=[
        pl.BlockSpec((NC, H1, BN, RW1), lambda i: (0, 0, i, 0)),
        pl.BlockSpec((H1, O1 + 1, C), lambda i: (0, 0, 0)),
        pl.BlockSpec((H1, C), lambda i: (0, 0)),
    ],
    out_specs=[
        pl.BlockSpec((BN, C), lambda i: (i, 0)),
        pl.BlockSpec((H1, BN), lambda i: (0, i)),
    ],
    out_shape=[
        jax.ShapeDtypeStruct((NP, C), jnp.float32),
        jax.ShapeDtypeStruct((H1, NP), jnp.float32),
    ],
)


def _k5_body(acc_ref, out_ref):
    y = acc_ref[0] + acc_ref[1]
    inv = 1.0 / (y[0, :, 8] + 1e-16)
    x2 = jnp.concatenate([y[0, :, 0:O1], y[1, :, 0:O1]], axis=1)
    x2 = x2 * inv[:, None]
    m = jnp.max(x2, axis=1, keepdims=True)
    z = x2 - m
    out_ref[...] = z - jnp.log(jnp.sum(jnp.exp(z), axis=1, keepdims=True))


_k5 = pl.pallas_call(
    _k5_body,
    grid=(NP // BN,),
    in_specs=[pl.BlockSpec((NC, 2, BN, RW2), lambda i: (0, 0, i, 0))],
    out_specs=pl.BlockSpec((BN, C), lambda i: (i, 0)),
    out_shape=jax.ShapeDtypeStruct((NP, C), jnp.float32),
)


def kernel(x, edge_index, w_mul, W1, a1_src, a1_dst, W2, a2_src, a2_dst):
    xp = jnp.pad(x, ((0, NP - N), (0, 0)))

    h1hm, ssrct, sdstt = _k1(xp, W1, a1_src, a1_dst)
    z1 = jnp.zeros((H1 * NP // NS, RW1), jnp.float32)
    acc1 = _get_edge_kernel("l1")(edge_index, w_mul,
                                  h1hm.reshape(H1 * NP, O1), ssrct, sdstt, z1)

    w2r = W2.reshape(H1, O1 + 1, C)
    a2 = jnp.concatenate(
        [a2_src, a2_dst, jnp.zeros((H1 - 2, C), jnp.float32)], axis=0)
    h2, s2 = _k3(acc1.reshape(NC, H1, NP, RW1), w2r, a2)

    s2src = jnp.broadcast_to(s2[0:1], (2, NP))
    s2dst = jnp.broadcast_to(s2[1:2], (2, NP))
    z2 = jnp.zeros((2 * NP // NS, RW2), jnp.float32)
    acc2 = _get_edge_kernel("l2")(edge_index, w_mul, h2, s2src, s2dst, z2)

    out = _k5(acc2.reshape(NC, 2, NP, RW2))
    return out[:N]
